# Initial kernel scaffold; baseline (speedup 1.0000x reference)
import jax
import jax.numpy as jnp
from jax.experimental import pallas as pl


def kernel(x):
    raise NotImplementedError("write your pallas kernel here")


# speedup vs baseline: 5.5714x; 5.5714x over previous
import jax
import jax.numpy as jnp
from jax.experimental import pallas as pl
from jax.experimental.pallas import tpu as pltpu


def _upsample_kernel(x_ref, ct2_ref, o_ref):
    o_ref[...] = jnp.dot(
        x_ref[...], ct2_ref[...], preferred_element_type=jnp.float32
    ).astype(o_ref.dtype)


def _upsample2x_rows(x2d, block_rows):
    rows, w_in = x2d.shape
    w_up = 2 * w_in
    w_out = 2 * w_up
    grid = (rows // block_rows,)

    col_src = (jnp.arange(w_out, dtype=jnp.int32) % w_up) // 2
    ct2 = (jnp.arange(w_in, dtype=jnp.int32)[:, None] == col_src[None, :])
    ct2 = ct2.astype(jnp.float32)

    return pl.pallas_call(
        _upsample_kernel,
        out_shape=jax.ShapeDtypeStruct((rows, w_out), x2d.dtype),
        grid_spec=pltpu.PrefetchScalarGridSpec(
            num_scalar_prefetch=0,
            grid=grid,
            in_specs=[
                pl.BlockSpec((block_rows, w_in), lambda i: (i, 0)),
                pl.BlockSpec((w_in, w_out), lambda i: (0, 0)),
            ],
            out_specs=pl.BlockSpec((block_rows, w_out), lambda i: (i, 0)),
        ),
        compiler_params=pltpu.CompilerParams(
            dimension_semantics=("parallel",),
            vmem_limit_bytes=64 * 1024 * 1024,
        ),
        cost_estimate=pl.CostEstimate(
            flops=2 * rows * w_in * w_out,
            transcendentals=0,
            bytes_accessed=rows * (w_in + w_out) * x2d.dtype.itemsize,
        ),
    )(x2d, ct2)


@jax.jit
def kernel(x):
    b, c, h, w = x.shape
    x2d = x.reshape(b * c * h, w)
    out256 = _upsample2x_rows(x2d, block_rows=2048)
    return out256.reshape(b, c, 2 * h, 2 * w)


# speedup vs baseline: 14.5669x; 2.6146x over previous
import jax
import jax.numpy as jnp
from jax.experimental import pallas as pl
from jax.experimental.pallas import tpu as pltpu


def _upsample_kernel(x_ref, ct2_ref, o_ref):
    y = jnp.dot(
        x_ref[...], ct2_ref[...], preferred_element_type=jnp.float32
    ).astype(o_ref.dtype)
    o_ref[::2, :] = y
    o_ref[1::2, :] = y


def _upsample2x_rows(x2d, block_rows):
    rows, w_in = x2d.shape
    w_out = 2 * w_in
    grid = (rows // block_rows,)

    col_src = jnp.arange(w_out, dtype=jnp.int32) // 2
    ct2 = (jnp.arange(w_in, dtype=jnp.int32)[:, None] == col_src[None, :])
    ct2 = ct2.astype(jnp.float32)

    return pl.pallas_call(
        _upsample_kernel,
        out_shape=jax.ShapeDtypeStruct((2 * rows, w_out), x2d.dtype),
        grid_spec=pltpu.PrefetchScalarGridSpec(
            num_scalar_prefetch=0,
            grid=grid,
            in_specs=[
                pl.BlockSpec((block_rows, w_in), lambda i: (i, 0)),
                pl.BlockSpec((w_in, w_out), lambda i: (0, 0)),
            ],
            out_specs=pl.BlockSpec((2 * block_rows, w_out), lambda i: (i, 0)),
        ),
        compiler_params=pltpu.CompilerParams(
            dimension_semantics=("parallel",),
            vmem_limit_bytes=64 * 1024 * 1024,
        ),
        cost_estimate=pl.CostEstimate(
            flops=2 * rows * w_in * w_out,
            transcendentals=0,
            bytes_accessed=rows * (w_in + 4 * w_in) * x2d.dtype.itemsize,
        ),
    )(x2d, ct2)


@jax.jit
def kernel(x):
    b, c, h, w = x.shape
    x2d = x.reshape(b * c * h, w)
    out2d = _upsample2x_rows(x2d, block_rows=2048)
    return out2d.reshape(b, c, 2 * h, 2 * w)


# speedup vs baseline: 17.9277x; 1.2307x over previous
import jax
import jax.numpy as jnp
from jax.experimental import pallas as pl
from jax.experimental.pallas import tpu as pltpu


def _upsample_kernel(x_ref, ct2_ref, o_ref):
    y = jnp.dot(
        x_ref[...], ct2_ref[...], preferred_element_type=jnp.float32
    ).astype(o_ref.dtype)
    o_ref[::2, :] = y
    o_ref[1::2, :] = y


def _upsample2x_rows(x2d, block_rows):
    rows, w_in = x2d.shape
    w_out = 2 * w_in
    grid = (rows // block_rows,)

    col_src = jnp.arange(w_out, dtype=jnp.int32) // 2
    ct2 = (jnp.arange(w_in, dtype=jnp.int32)[:, None] == col_src[None, :])
    ct2 = ct2.astype(jnp.float32)

    return pl.pallas_call(
        _upsample_kernel,
        out_shape=jax.ShapeDtypeStruct((2 * rows, w_out), x2d.dtype),
        grid_spec=pltpu.PrefetchScalarGridSpec(
            num_scalar_prefetch=0,
            grid=grid,
            in_specs=[
                pl.BlockSpec((block_rows, w_in), lambda i: (i, 0)),
                pl.BlockSpec((w_in, w_out), lambda i: (0, 0)),
            ],
            out_specs=pl.BlockSpec((2 * block_rows, w_out), lambda i: (i, 0)),
        ),
        compiler_params=pltpu.CompilerParams(
            dimension_semantics=("parallel",),
            vmem_limit_bytes=64 * 1024 * 1024,
        ),
        cost_estimate=pl.CostEstimate(
            flops=2 * rows * w_in * w_out,
            transcendentals=0,
            bytes_accessed=rows * (w_in + 4 * w_in) * x2d.dtype.itemsize,
        ),
    )(x2d, ct2)


@jax.jit
def kernel(x):
    b, c, h, w = x.shape
    x2d = x.reshape(b * c * h, w)
    out2d = _upsample2x_rows(x2d, block_rows=4096)
    return out2d.reshape(b, c, 2 * h, 2 * w)


# speedup vs baseline: 18.8180x; 1.0497x over previous
import jax
import jax.numpy as jnp
from jax.experimental import pallas as pl
from jax.experimental.pallas import tpu as pltpu


def _upsample_kernel(x_ref, ct2_ref, o_ref):
    y = jnp.dot(
        x_ref[...], ct2_ref[...], preferred_element_type=jnp.float32
    ).astype(o_ref.dtype)
    o_ref[::2, :] = y
    o_ref[1::2, :] = y


def _upsample2x_rows(x2d, block_rows):
    rows, w_in = x2d.shape
    w_out = 2 * w_in
    grid = (rows // block_rows,)

    col_src = jnp.arange(w_out, dtype=jnp.int32) // 2
    ct2 = (jnp.arange(w_in, dtype=jnp.int32)[:, None] == col_src[None, :])
    ct2 = ct2.astype(jnp.float32)

    return pl.pallas_call(
        _upsample_kernel,
        out_shape=jax.ShapeDtypeStruct((2 * rows, w_out), x2d.dtype),
        grid_spec=pltpu.PrefetchScalarGridSpec(
            num_scalar_prefetch=0,
            grid=grid,
            in_specs=[
                pl.BlockSpec((block_rows, w_in), lambda i: (i, 0)),
                pl.BlockSpec((w_in, w_out), lambda i: (0, 0)),
            ],
            out_specs=pl.BlockSpec((2 * block_rows, w_out), lambda i: (i, 0)),
        ),
        compiler_params=pltpu.CompilerParams(
            dimension_semantics=("parallel",),
            vmem_limit_bytes=64 * 1024 * 1024,
        ),
        cost_estimate=pl.CostEstimate(
            flops=2 * rows * w_in * w_out,
            transcendentals=0,
            bytes_accessed=rows * (w_in + 4 * w_in) * x2d.dtype.itemsize,
        ),
    )(x2d, ct2)


@jax.jit
def kernel(x):
    b, c, h, w = x.shape
    x2d = x.reshape(b * c * h, w)
    out2d = _upsample2x_rows(x2d, block_rows=8192)
    return out2d.reshape(b, c, 2 * h, 2 * w)


# speedup vs baseline: 19.3689x; 1.0293x over previous
import jax
import jax.numpy as jnp
from jax.experimental import pallas as pl
from jax.experimental.pallas import tpu as pltpu


def _upsample_kernel(x_ref, ct2_ref, o_ref):
    y = jnp.dot(
        x_ref[...], ct2_ref[...], preferred_element_type=jnp.float32
    ).astype(o_ref.dtype)
    o_ref[::2, :] = y
    o_ref[1::2, :] = y


def _upsample2x_rows(x2d, block_rows):
    rows, w_in = x2d.shape
    w_out = 2 * w_in
    grid = (rows // block_rows,)

    col_src = jnp.arange(w_out, dtype=jnp.int32) // 2
    ct2 = (jnp.arange(w_in, dtype=jnp.int32)[:, None] == col_src[None, :])
    ct2 = ct2.astype(jnp.float32)

    return pl.pallas_call(
        _upsample_kernel,
        out_shape=jax.ShapeDtypeStruct((2 * rows, w_out), x2d.dtype),
        grid_spec=pltpu.PrefetchScalarGridSpec(
            num_scalar_prefetch=0,
            grid=grid,
            in_specs=[
                pl.BlockSpec((block_rows, w_in), lambda i: (i, 0)),
                pl.BlockSpec((w_in, w_out), lambda i: (0, 0)),
            ],
            out_specs=pl.BlockSpec((2 * block_rows, w_out), lambda i: (i, 0)),
        ),
        compiler_params=pltpu.CompilerParams(
            dimension_semantics=("parallel",),
            vmem_limit_bytes=64 * 1024 * 1024,
        ),
        cost_estimate=pl.CostEstimate(
            flops=2 * rows * w_in * w_out,
            transcendentals=0,
            bytes_accessed=rows * (w_in + 4 * w_in) * x2d.dtype.itemsize,
        ),
    )(x2d, ct2)


@jax.jit
def kernel(x):
    b, c, h, w = x.shape
    x2d = x.reshape(b * c * h, w)
    out2d = _upsample2x_rows(x2d, block_rows=16384)
    return out2d.reshape(b, c, 2 * h, 2 * w)
